# Initial kernel scaffold; baseline (speedup 1.0000x reference)
import jax
import jax.numpy as jnp
from jax.experimental import pallas as pl


def kernel(x, y, stds, covar_factors):
    raise NotImplementedError("write your pallas kernel here")


# speedup vs baseline: 3.5537x; 3.5537x over previous
import functools

import jax
import jax.numpy as jnp
from jax import lax
from jax.experimental import pallas as pl
from jax.experimental.pallas import tpu as pltpu
from jax.experimental.pallas import tpu_sc as plsc

NB_CAT = 1000
RANK = 16
NF = 26
B = 16384
L = 16
NC = 2
NS = 16
NW = NC * NS
BPW = B // NW


def _body(x_hbm, y_hbm, stds_hbm, cf_hbm, out_hbm, xt, yt, sv, tab, acc):
    wid = lax.axis_index("s") * NC + lax.axis_index("c")
    pltpu.sync_copy(x_hbm.at[wid], xt)
    pltpu.sync_copy(y_hbm.at[wid], yt)
    pltpu.sync_copy(stds_hbm, sv)
    zero = jnp.zeros((L,), jnp.float32)
    for f in range(NF):
        pltpu.sync_copy(cf_hbm.at[f], tab)
        soff = jnp.full((L,), f * NB_CAT, jnp.int32)

        def rowgroup(g, _, f=f, soff=soff):
            sl = pl.ds(g * L, L)
            xv = xt[f, sl]
            yv = yt[f, sl]
            s = plsc.load_gather(sv, [soff + xv])
            a = jnp.where(xv == yv, s * s, zero)
            bx = xv * RANK
            by = yv * RANK
            gx = plsc.load_gather(tab, [bx])
            gy = plsc.load_gather(tab, [by])
            a = a + gx * gy
            for r in range(1, RANK):
                gx = plsc.load_gather(tab, [bx + r])
                gy = plsc.load_gather(tab, [by + r])
                a = a + gx * gy
            if f == 0:
                acc[sl] = a
            else:
                acc[sl] = acc[sl] + a
            return 0

        lax.fori_loop(0, BPW // L, rowgroup, 0)
    pltpu.sync_copy(acc, out_hbm.at[pl.ds(wid * BPW, BPW)])


@jax.jit
def kernel(x, y, stds, covar_factors):
    x_r = x.reshape(NW, BPW, NF).transpose(0, 2, 1)
    y_r = y.reshape(NW, BPW, NF).transpose(0, 2, 1)
    stds_f = stds.reshape(NF * NB_CAT)
    cf_r = covar_factors.reshape(NF, NB_CAT * RANK)

    mesh = plsc.VectorSubcoreMesh(core_axis_name="c", subcore_axis_name="s")
    run = pl.kernel(
        _body,
        out_type=jax.ShapeDtypeStruct((B,), jnp.float32),
        mesh=mesh,
        compiler_params=pltpu.CompilerParams(needs_layout_passes=False),
        scratch_types=[
            pltpu.VMEM((NF, BPW), jnp.int32),
            pltpu.VMEM((NF, BPW), jnp.int32),
            pltpu.VMEM((NF * NB_CAT,), jnp.float32),
            pltpu.VMEM((NB_CAT * RANK,), jnp.float32),
            pltpu.VMEM((BPW,), jnp.float32),
        ],
    )
    return run(x_r, y_r, stds_f, cf_r)


# speedup vs baseline: 5.0708x; 1.4269x over previous
import functools

import jax
import jax.numpy as jnp
from jax import lax
from jax.experimental import pallas as pl
from jax.experimental.pallas import tpu as pltpu
from jax.experimental.pallas import tpu_sc as plsc

NB_CAT = 1000
RANK = 16
NF = 26
B = 16384
L = 16
NC = 2
NS = 16
NW = NC * NS
BPW = B // NW


def _field_pass(f, tab_b, xt, yt, sv, acc):
    zero = jnp.zeros((L,), jnp.float32)
    soff = jnp.full((L,), f * NB_CAT, jnp.int32)

    def rowgroup(g, _):
        sl = pl.ds(g * L, L)
        xv = xt[f, sl]
        yv = yt[f, sl]
        s = plsc.load_gather(sv, [soff + xv])
        d = jnp.where(xv == yv, s * s, zero)
        bx = xv * RANK
        by = yv * RANK
        part = [d, zero, zero, zero]
        for r in range(RANK):
            gx = plsc.load_gather(tab_b, [bx + r] if r else [bx])
            gy = plsc.load_gather(tab_b, [by + r] if r else [by])
            part[r % 4] = part[r % 4] + gx * gy
        a = (part[0] + part[1]) + (part[2] + part[3])
        if f == 0:
            acc[sl] = a
        else:
            acc[sl] = acc[sl] + a
        return 0

    lax.fori_loop(0, BPW // L, rowgroup, 0)


def _body(x_hbm, y_hbm, stds_hbm, cf_hbm, out_hbm, xt, yt, sv, tab0, tab1,
          acc, sem0, sem1):
    wid = lax.axis_index("s") * NC + lax.axis_index("c")
    sems = (sem0, sem1)
    tabs = (tab0, tab1)
    desc = [None, None]
    desc[0] = pltpu.async_copy(cf_hbm.at[0], tab0, sems[0])
    pltpu.sync_copy(x_hbm.at[wid], xt)
    pltpu.sync_copy(y_hbm.at[wid], yt)
    pltpu.sync_copy(stds_hbm, sv)
    for f in range(NF):
        b = f % 2
        if f + 1 < NF:
            desc[1 - b] = pltpu.async_copy(
                cf_hbm.at[f + 1], tabs[1 - b], sems[1 - b])
        desc[b].wait()
        _field_pass(f, tabs[b], xt, yt, sv, acc)
    pltpu.sync_copy(acc, out_hbm.at[pl.ds(wid * BPW, BPW)])


@jax.jit
def kernel(x, y, stds, covar_factors):
    x_r = x.reshape(NW, BPW, NF).transpose(0, 2, 1)
    y_r = y.reshape(NW, BPW, NF).transpose(0, 2, 1)
    stds_f = stds.reshape(NF * NB_CAT)
    cf_r = covar_factors.reshape(NF, NB_CAT * RANK)

    mesh = plsc.VectorSubcoreMesh(core_axis_name="c", subcore_axis_name="s")
    run = pl.kernel(
        _body,
        out_type=jax.ShapeDtypeStruct((B,), jnp.float32),
        mesh=mesh,
        compiler_params=pltpu.CompilerParams(needs_layout_passes=False),
        scratch_types=[
            pltpu.VMEM((NF, BPW), jnp.int32),
            pltpu.VMEM((NF, BPW), jnp.int32),
            pltpu.VMEM((NF * NB_CAT,), jnp.float32),
            pltpu.VMEM((NB_CAT * RANK,), jnp.float32),
            pltpu.VMEM((NB_CAT * RANK,), jnp.float32),
            pltpu.VMEM((BPW,), jnp.float32),
            pltpu.SemaphoreType.DMA,
            pltpu.SemaphoreType.DMA,
        ],
    )
    return run(x_r, y_r, stds_f, cf_r)
